# baseline (speedup 1.0000x reference)
import functools

import jax
import jax.numpy as jnp
from jax import lax
from jax.experimental import pallas as pl
from jax.experimental.pallas import tpu as pltpu
from jax.experimental.pallas import tpu_sc as plsc

NC, NS, L = 2, 16, 16
NW = NC * NS
B, DIM, H, W = 32, 64, 128, 128
HW = H * W
M = 500
MP = 512
G = 8
NG = DIM // G
GMP = G * MP


@functools.partial(
    pl.kernel,
    out_type=(
        jax.ShapeDtypeStruct((NW, L), jnp.float32),
        jax.ShapeDtypeStruct((NW, L), jnp.float32),
    ),
    mesh=plsc.VectorSubcoreMesh(
        core_axis_name="c", subcore_axis_name="s",
        num_cores=NC, num_subcores=NS),
    compiler_params=pltpu.CompilerParams(
        needs_layout_passes=False, use_tc_tiling_on_sc=False),
    scratch_types=[
        pltpu.VMEM((MP,), jnp.int32),
        pltpu.VMEM((MP,), jnp.int32),
        pltpu.VMEM((GMP,), jnp.int32),
        pltpu.VMEM((GMP,), jnp.float32),
        pltpu.VMEM((DIM * MP,), jnp.float32),
        pltpu.VMEM((2, GMP), jnp.float32),
        pltpu.VMEM((L,), jnp.float32),
        pltpu.VMEM((L,), jnp.float32),
        pltpu.SemaphoreType.DMA,
        pltpu.SemaphoreType.DMA,
        pltpu.SemaphoreType.DMA,
    ],
)
def _sc_loss(outel, indf, maskf, tgtf, loss_out, num_out,
             ind_v, mski_v, cix_v, mask_v, tgt_v, prd_v, acc_v, nacc_v,
             sem0, sem1, semt):
    w = lax.axis_index("s") * NC + lax.axis_index("c")
    base = w * DIM

    zi = jnp.zeros((L,), jnp.int32)
    ind_v[pl.ds(MP - L, L)] = zi
    mski_v[pl.ds(MP - L, L)] = zi

    tgt_cp = pltpu.async_copy(tgtf.at[w], tgt_v, semt)
    pltpu.sync_copy(indf.at[w], ind_v.at[pl.ds(0, M)])
    pltpu.sync_copy(maskf.at[w], mski_v.at[pl.ds(0, M)])

    def split_body(j, nacc):
        iv = ind_v[pl.ds(j * L, L)]
        mk = mski_v[pl.ds(j * L, L)].astype(jnp.float32)
        mask_v[pl.ds(j * L, L)] = mk
        for k in range(G):
            cix_v[pl.ds(k * MP + j * L, L)] = iv + (k * HW)
            if k:
                mask_v[pl.ds(k * MP + j * L, L)] = mk
        return nacc + mk

    nacc = lax.fori_loop(0, MP // L, split_body,
                         jnp.zeros((L,), jnp.float32), unroll=2)

    def gather_group(g, buf, sem):
        table = outel.at[pl.ds((base + g * G) * HW, G * HW)]
        return pltpu.async_copy(table.at[cix_v], prd_v.at[buf], sem)

    def wait_group(g, buf, sem):
        table = outel.at[pl.ds((base + g * G) * HW, G * HW)]
        pltpu.make_async_copy(table.at[cix_v], prd_v.at[buf], sem).wait()

    def compute_group(buf, g, acc):
        tbase = g * GMP

        def m_body(q, acc):
            p = prd_v[buf, pl.ds(q * L, L)]
            t = tgt_v[pl.ds(tbase + q * L, L)]
            mk = mask_v[pl.ds(q * L, L)]
            d = (p - t) * mk
            a = jnp.abs(d)
            m1 = jnp.minimum(a, 1.0)
            return acc + (0.5 * m1 * m1 - 1.0 + jnp.maximum(a, 1.0))

        return lax.fori_loop(0, GMP // L, m_body, acc, unroll=4)

    gather_group(0, 0, sem0)
    gather_group(1, 1, sem1)
    tgt_cp.wait()

    def step(i, acc):
        g0 = 2 * i
        wait_group(g0, 0, sem0)
        acc = compute_group(0, g0, acc)

        @pl.when(i < NG // 2 - 1)
        def _():
            gather_group(g0 + 2, 0, sem0)

        wait_group(g0 + 1, 1, sem1)
        acc = compute_group(1, g0 + 1, acc)

        @pl.when(i < NG // 2 - 1)
        def _():
            gather_group(g0 + 3, 1, sem1)

        return acc

    acc = lax.fori_loop(0, NG // 2, step, jnp.zeros((L,), jnp.float32))

    acc_v[...] = acc
    nacc_v[...] = nacc
    pltpu.sync_copy(acc_v, loss_out.at[w])
    pltpu.sync_copy(nacc_v, num_out.at[w])


def kernel(output, mask, ind, target):
    outel = output.reshape(B * DIM * HW)
    ind32 = ind.astype(jnp.int32)
    mask32 = mask.astype(jnp.int32)
    tgtT = jnp.pad(jnp.transpose(target, (0, 2, 1)),
                   ((0, 0), (0, 0), (0, MP - M)))
    tgtflat = tgtT.reshape(B, DIM * MP)
    loss_p, num_p = _sc_loss(outel, ind32, mask32, tgtflat)
    return jnp.sum(loss_p) / (jnp.sum(num_p) + 0.0001)


# speedup vs baseline: 1.0027x; 1.0027x over previous
import functools

import jax
import jax.numpy as jnp
from jax import lax
from jax.experimental import pallas as pl
from jax.experimental.pallas import tpu as pltpu
from jax.experimental.pallas import tpu_sc as plsc

NC, NS, L = 2, 16, 16
NW = NC * NS
B, DIM, H, W = 32, 64, 128, 128
HW = H * W
M = 500
MP = 512
G = 8
NG = DIM // G
GMP = G * MP


@functools.partial(
    pl.kernel,
    out_type=(
        jax.ShapeDtypeStruct((NW, L), jnp.float32),
        jax.ShapeDtypeStruct((NW, L), jnp.float32),
    ),
    mesh=plsc.VectorSubcoreMesh(
        core_axis_name="c", subcore_axis_name="s",
        num_cores=NC, num_subcores=NS),
    compiler_params=pltpu.CompilerParams(
        needs_layout_passes=False, use_tc_tiling_on_sc=False),
    scratch_types=[
        pltpu.VMEM((MP,), jnp.int32),
        pltpu.VMEM((MP,), jnp.int32),
        pltpu.VMEM((GMP,), jnp.int32),
        pltpu.VMEM((GMP,), jnp.float32),
        pltpu.VMEM((DIM * MP,), jnp.float32),
        pltpu.VMEM((NG, GMP), jnp.float32),
        pltpu.VMEM((L,), jnp.float32),
        pltpu.VMEM((L,), jnp.float32),
        pltpu.SemaphoreType.DMA,
        pltpu.SemaphoreType.DMA,
        pltpu.SemaphoreType.DMA,
        pltpu.SemaphoreType.DMA,
        pltpu.SemaphoreType.DMA,
        pltpu.SemaphoreType.DMA,
        pltpu.SemaphoreType.DMA,
        pltpu.SemaphoreType.DMA,
        pltpu.SemaphoreType.DMA,
    ],
)
def _sc_loss(outel, indf, maskf, tgtf, loss_out, num_out,
             ind_v, mski_v, cix_v, mask_v, tgt_v, prd_v, acc_v, nacc_v,
             sem0, sem1, sem2, sem3, sem4, sem5, sem6, sem7, semt):
    sems = (sem0, sem1, sem2, sem3, sem4, sem5, sem6, sem7)
    w = lax.axis_index("s") * NC + lax.axis_index("c")
    base = w * DIM

    zi = jnp.zeros((L,), jnp.int32)
    ind_v[pl.ds(MP - L, L)] = zi
    mski_v[pl.ds(MP - L, L)] = zi

    tgt_cp = pltpu.async_copy(tgtf.at[w], tgt_v, semt)
    pltpu.sync_copy(indf.at[w], ind_v.at[pl.ds(0, M)])
    pltpu.sync_copy(maskf.at[w], mski_v.at[pl.ds(0, M)])

    def split_body(j, nacc):
        iv = ind_v[pl.ds(j * L, L)]
        mk = mski_v[pl.ds(j * L, L)].astype(jnp.float32)
        mask_v[pl.ds(j * L, L)] = mk
        for k in range(G):
            cix_v[pl.ds(k * MP + j * L, L)] = iv + (k * HW)
            if k:
                mask_v[pl.ds(k * MP + j * L, L)] = mk
        return nacc + mk

    nacc = lax.fori_loop(0, MP // L, split_body,
                         jnp.zeros((L,), jnp.float32), unroll=2)

    def gather_group(g, buf, sem):
        table = outel.at[pl.ds((base + g * G) * HW, G * HW)]
        return pltpu.async_copy(table.at[cix_v], prd_v.at[buf], sem)

    def wait_group(g, buf, sem):
        table = outel.at[pl.ds((base + g * G) * HW, G * HW)]
        pltpu.make_async_copy(table.at[cix_v], prd_v.at[buf], sem).wait()

    def compute_group(buf, g, acc):
        tbase = g * GMP

        def m_body(q, acc):
            p = prd_v[buf, pl.ds(q * L, L)]
            t = tgt_v[pl.ds(tbase + q * L, L)]
            mk = mask_v[pl.ds(q * L, L)]
            d = (p - t) * mk
            a = jnp.abs(d)
            m1 = jnp.minimum(a, 1.0)
            return acc + (0.5 * m1 * m1 - 1.0 + jnp.maximum(a, 1.0))

        return lax.fori_loop(0, GMP // L, m_body, acc, unroll=4)

    for g in range(NG):
        gather_group(g, g, sems[g])
    tgt_cp.wait()

    acc = jnp.zeros((L,), jnp.float32)
    for g in range(NG):
        wait_group(g, g, sems[g])
        acc = compute_group(g, g, acc)

    acc_v[...] = acc
    nacc_v[...] = nacc
    pltpu.sync_copy(acc_v, loss_out.at[w])
    pltpu.sync_copy(nacc_v, num_out.at[w])


def kernel(output, mask, ind, target):
    outel = output.reshape(B * DIM * HW)
    ind32 = ind.astype(jnp.int32)
    mask32 = mask.astype(jnp.int32)
    tgtT = jnp.pad(jnp.transpose(target, (0, 2, 1)),
                   ((0, 0), (0, 0), (0, MP - M)))
    tgtflat = tgtT.reshape(B, DIM * MP)
    loss_p, num_p = _sc_loss(outel, ind32, mask32, tgtflat)
    return jnp.sum(loss_p) / (jnp.sum(num_p) + 0.0001)


# speedup vs baseline: 1.0208x; 1.0180x over previous
import functools

import jax
import jax.numpy as jnp
from jax import lax
from jax.experimental import pallas as pl
from jax.experimental.pallas import tpu as pltpu
from jax.experimental.pallas import tpu_sc as plsc

NC, NS, L = 2, 16, 16
NW = NC * NS
B, DIM, H, W = 32, 64, 128, 128
HW = H * W
M = 500
MP = 512
G = 8
NG = DIM // G
GMP = G * MP


@functools.partial(
    pl.kernel,
    out_type=jax.ShapeDtypeStruct((NW, 2, L), jnp.float32),
    mesh=plsc.VectorSubcoreMesh(
        core_axis_name="c", subcore_axis_name="s",
        num_cores=NC, num_subcores=NS),
    compiler_params=pltpu.CompilerParams(
        needs_layout_passes=False, use_tc_tiling_on_sc=False),
    scratch_types=[
        pltpu.VMEM((MP,), jnp.int32),
        pltpu.VMEM((MP,), jnp.int32),
        pltpu.VMEM((GMP,), jnp.int32),
        pltpu.VMEM((GMP,), jnp.float32),
        pltpu.VMEM((DIM * MP,), jnp.float32),
        pltpu.VMEM((NG, GMP), jnp.float32),
        pltpu.VMEM((2, L), jnp.float32),
        pltpu.SemaphoreType.DMA,
        pltpu.SemaphoreType.DMA,
        pltpu.SemaphoreType.DMA,
        pltpu.SemaphoreType.DMA,
        pltpu.SemaphoreType.DMA,
        pltpu.SemaphoreType.DMA,
        pltpu.SemaphoreType.DMA,
        pltpu.SemaphoreType.DMA,
        pltpu.SemaphoreType.DMA,
    ],
)
def _sc_loss(outel, indf, maskf, tgtf, out,
             ind_v, mski_v, cix_v, mask_v, tgt_v, prd_v, acc_v,
             sem0, sem1, sem2, sem3, sem4, sem5, sem6, sem7, semt):
    sems = (sem0, sem1, sem2, sem3, sem4, sem5, sem6, sem7)
    w = lax.axis_index("s") * NC + lax.axis_index("c")
    base = w * DIM

    zi = jnp.zeros((L,), jnp.int32)
    ind_v[pl.ds(MP - L, L)] = zi
    mski_v[pl.ds(MP - L, L)] = zi

    tgt_cp = pltpu.async_copy(tgtf.at[w], tgt_v, semt)
    pltpu.sync_copy(indf.at[w], ind_v.at[pl.ds(0, M)])
    pltpu.sync_copy(maskf.at[w], mski_v.at[pl.ds(0, M)])

    def split_body(j, nacc):
        iv = ind_v[pl.ds(j * L, L)]
        mk = mski_v[pl.ds(j * L, L)].astype(jnp.float32)
        mask_v[pl.ds(j * L, L)] = mk
        for k in range(G):
            cix_v[pl.ds(k * MP + j * L, L)] = iv + (k * HW)
            if k:
                mask_v[pl.ds(k * MP + j * L, L)] = mk
        return nacc + mk

    nacc = lax.fori_loop(0, MP // L, split_body,
                         jnp.zeros((L,), jnp.float32), unroll=2)

    def gather_group(g, buf, sem):
        table = outel.at[pl.ds((base + g * G) * HW, G * HW)]
        return pltpu.async_copy(table.at[cix_v], prd_v.at[buf], sem)

    def wait_group(g, buf, sem):
        table = outel.at[pl.ds((base + g * G) * HW, G * HW)]
        pltpu.make_async_copy(table.at[cix_v], prd_v.at[buf], sem).wait()

    def compute_group(buf, g, acc):
        tbase = g * GMP

        def m_body(q, acc):
            p = prd_v[buf, pl.ds(q * L, L)]
            t = tgt_v[pl.ds(tbase + q * L, L)]
            mk = mask_v[pl.ds(q * L, L)]
            d = (p - t) * mk
            a = jnp.abs(d)
            m1 = jnp.minimum(a, 1.0)
            return acc + (0.5 * m1 * m1 - 1.0 + jnp.maximum(a, 1.0))

        return lax.fori_loop(0, GMP // L, m_body, acc, unroll=4)

    for g in range(NG):
        gather_group(g, g, sems[g])
    tgt_cp.wait()

    acc = jnp.zeros((L,), jnp.float32)
    for g in range(NG):
        wait_group(g, g, sems[g])
        acc = compute_group(g, g, acc)

    acc_v[0, :] = acc
    acc_v[1, :] = nacc
    pltpu.sync_copy(acc_v, out.at[w])


def kernel(output, mask, ind, target):
    outel = output.reshape(B * DIM * HW)
    ind32 = ind.astype(jnp.int32)
    mask32 = mask.astype(jnp.int32)
    tgtT = jnp.pad(jnp.transpose(target, (0, 2, 1)),
                   ((0, 0), (0, 0), (0, MP - M)))
    tgtflat = tgtT.reshape(B, DIM * MP)
    out = _sc_loss(outel, ind32, mask32, tgtflat)
    return jnp.sum(out[:, 0]) / (jnp.sum(out[:, 1]) + 0.0001)
